# baseline (speedup 1.0000x reference)
import jax
import jax.numpy as jnp
from jax.experimental import pallas as pl
from jax.experimental.pallas import tpu as pltpu

N_PSI = 24
N_CONES = 192
N_SO3 = N_CONES * N_PSI
BLOCK_B = 1024


def _fused_kernel(pred_ref, grid_ref, scores_ref, med_ref, mad_ref, out_ref):
    bb = pred_ref.shape[2]
    p9 = jnp.concatenate([pred_ref[0], pred_ref[1], pred_ref[2]], axis=0)
    g9 = jnp.concatenate([grid_ref[0], grid_ref[1], grid_ref[2]], axis=0)
    simT = jax.lax.dot_general(
        g9, p9, (((0,), (0,)), ((), ())),
        preferred_element_type=jnp.float32)
    m8 = jnp.max(simT.reshape(N_CONES, 3, 8, bb), axis=1)
    gmax = jnp.max(m8, axis=(0, 1), keepdims=True)
    cidx = jax.lax.broadcasted_iota(jnp.int32, (N_CONES, 1, 1), 0)
    cone = jnp.min(jnp.where(m8 == gmax, cidx, N_CONES),
                   axis=(0, 1), keepdims=True)
    onehotT = (cone[0] == jax.lax.broadcasted_iota(
        jnp.int32, (N_CONES, 1), 0)).astype(jnp.float32)
    stats = jnp.concatenate([med_ref[...], mad_ref[...]], axis=0)
    st = jnp.dot(stats, onehotT, preferred_element_type=jnp.float32)
    out_ref[...] = (scores_ref[...] - st[0:1, :]) / st[1:2, :]


@jax.jit
def kernel(pred_rotmats, scores, grid_rotmats, medians, mads):
    b = pred_rotmats.shape[0]
    predT = pred_rotmats.transpose(1, 2, 0)
    gridT = grid_rotmats.transpose(1, 2, 0)

    out = pl.pallas_call(
        _fused_kernel,
        grid=(b // BLOCK_B,),
        in_specs=[
            pl.BlockSpec((3, 3, BLOCK_B), lambda i: (0, 0, i)),
            pl.BlockSpec((3, 3, N_SO3), lambda i: (0, 0, 0)),
            pl.BlockSpec((1, BLOCK_B), lambda i: (0, i)),
            pl.BlockSpec((1, N_CONES), lambda i: (0, 0)),
            pl.BlockSpec((1, N_CONES), lambda i: (0, 0)),
        ],
        out_specs=pl.BlockSpec((1, BLOCK_B), lambda i: (0, i)),
        out_shape=jax.ShapeDtypeStruct((1, b), jnp.float32),
        compiler_params=pltpu.CompilerParams(
            dimension_semantics=("parallel",)),
    )(predT, gridT, scores.reshape(1, b),
      medians.reshape(1, N_CONES), mads.reshape(1, N_CONES))
    return out.reshape(b)


# speedup vs baseline: 1.1441x; 1.1441x over previous
import jax
import jax.numpy as jnp
from jax.experimental import pallas as pl
from jax.experimental.pallas import tpu as pltpu

N_PSI = 24
N_CONES = 192
N_SO3 = N_CONES * N_PSI
BLOCK_B = 2048


def _fused_kernel(pred_ref, grid_ref, scores_ref, med_ref, mad_ref, out_ref):
    bb = pred_ref.shape[2]
    p9 = jnp.concatenate([pred_ref[0], pred_ref[1], pred_ref[2]], axis=0)
    g9 = jnp.concatenate([grid_ref[0], grid_ref[1], grid_ref[2]], axis=0)
    simT = jax.lax.dot_general(
        g9, p9, (((0,), (0,)), ((), ())),
        preferred_element_type=jnp.float32)
    m8 = jnp.max(simT.reshape(N_CONES, 3, 8, bb), axis=1)
    gmax = jnp.max(m8, axis=(0, 1), keepdims=True)
    cidx = jax.lax.broadcasted_iota(jnp.int32, (N_CONES, 1, 1), 0)
    cone = jnp.min(jnp.where(m8 == gmax, cidx, N_CONES),
                   axis=(0, 1), keepdims=True)
    onehotT = (cone[0] == jax.lax.broadcasted_iota(
        jnp.int32, (N_CONES, 1), 0)).astype(jnp.float32)
    stats = jnp.concatenate([med_ref[...], mad_ref[...]], axis=0)
    st = jnp.dot(stats, onehotT, preferred_element_type=jnp.float32)
    out_ref[...] = (scores_ref[...] - st[0:1, :]) / st[1:2, :]


@jax.jit
def kernel(pred_rotmats, scores, grid_rotmats, medians, mads):
    b = pred_rotmats.shape[0]
    predT = pred_rotmats.transpose(1, 2, 0)
    gridT = grid_rotmats.transpose(1, 2, 0)

    out = pl.pallas_call(
        _fused_kernel,
        grid=(b // BLOCK_B,),
        in_specs=[
            pl.BlockSpec((3, 3, BLOCK_B), lambda i: (0, 0, i)),
            pl.BlockSpec((3, 3, N_SO3), lambda i: (0, 0, 0)),
            pl.BlockSpec((1, BLOCK_B), lambda i: (0, i)),
            pl.BlockSpec((1, N_CONES), lambda i: (0, 0)),
            pl.BlockSpec((1, N_CONES), lambda i: (0, 0)),
        ],
        out_specs=pl.BlockSpec((1, BLOCK_B), lambda i: (0, i)),
        out_shape=jax.ShapeDtypeStruct((1, b), jnp.float32),
        compiler_params=pltpu.CompilerParams(
            dimension_semantics=("parallel",)),
    )(predT, gridT, scores.reshape(1, b),
      medians.reshape(1, N_CONES), mads.reshape(1, N_CONES))
    return out.reshape(b)
